# speedup vs baseline: 2.3668x; 2.3668x over previous
import functools
import jax
import jax.numpy as jnp
from jax.experimental import pallas as pl
from jax.experimental.pallas import tpu as pltpu

N_NODES = 10000
ROW_BLK = 2000


def _mm_scale_body(x_ref, w_ref, dinv_ref, out_ref):
    h = jnp.dot(x_ref[...], w_ref[...], preferred_element_type=jnp.float32)
    out_ref[...] = h * dinv_ref[...]


def _mm_scale(x, w, dinv):
    n, fin = x.shape
    fout = w.shape[1]
    grid = n // ROW_BLK
    return pl.pallas_call(
        _mm_scale_body,
        grid=(grid,),
        in_specs=[
            pl.BlockSpec((ROW_BLK, fin), lambda i: (i, 0)),
            pl.BlockSpec((fin, fout), lambda i: (0, 0)),
            pl.BlockSpec((ROW_BLK, 1), lambda i: (i, 0)),
        ],
        out_specs=pl.BlockSpec((ROW_BLK, fout), lambda i: (i, 0)),
        out_shape=jax.ShapeDtypeStruct((n, fout), jnp.float32),
    )(x, w, dinv)


def _combine_mm_body(s_ref, g_ref, dinv_ref, b_ref, w_ref, out_ref):
    x = jnp.maximum((s_ref[...] + g_ref[...]) * dinv_ref[...] + b_ref[...], 0.0)
    h = jnp.dot(x, w_ref[...], preferred_element_type=jnp.float32)
    out_ref[...] = h * dinv_ref[...]


def _combine_mm(s, g, dinv, b, w):
    n, fin = g.shape
    fout = w.shape[1]
    grid = n // ROW_BLK
    return pl.pallas_call(
        _combine_mm_body,
        grid=(grid,),
        in_specs=[
            pl.BlockSpec((ROW_BLK, fin), lambda i: (i, 0)),
            pl.BlockSpec((ROW_BLK, fin), lambda i: (i, 0)),
            pl.BlockSpec((ROW_BLK, 1), lambda i: (i, 0)),
            pl.BlockSpec((1, fin), lambda i: (0, 0)),
            pl.BlockSpec((fin, fout), lambda i: (0, 0)),
        ],
        out_specs=pl.BlockSpec((ROW_BLK, fout), lambda i: (i, 0)),
        out_shape=jax.ShapeDtypeStruct((n, fout), jnp.float32),
    )(s, g, dinv, b.reshape(1, fin), w)


def _combine_final_body(s_ref, g_ref, dinv_ref, b_ref, out_ref):
    out_ref[...] = jnp.maximum(
        (s_ref[...] + g_ref[...]) * dinv_ref[...] + b_ref[...], 0.0
    )


def _combine_final(s, g, dinv, b):
    n, f = g.shape
    grid = n // ROW_BLK
    return pl.pallas_call(
        _combine_final_body,
        grid=(grid,),
        in_specs=[
            pl.BlockSpec((ROW_BLK, f), lambda i: (i, 0)),
            pl.BlockSpec((ROW_BLK, f), lambda i: (i, 0)),
            pl.BlockSpec((ROW_BLK, 1), lambda i: (i, 0)),
            pl.BlockSpec((1, f), lambda i: (0, 0)),
        ],
        out_specs=pl.BlockSpec((ROW_BLK, f), lambda i: (i, 0)),
        out_shape=jax.ShapeDtypeStruct((n, f), jnp.float32),
    )(s, g, dinv, b.reshape(1, f))


def kernel(v, edge_index, W1, b1, W2, b2, W3, b3):
    src = edge_index[0]
    dst = edge_index[1]
    deg = jax.ops.segment_sum(
        jnp.ones_like(dst, dtype=jnp.float32), dst, num_segments=N_NODES
    ) + 1.0
    dinv = jax.lax.rsqrt(deg).reshape(N_NODES, 1)

    def scatter(g):
        return jax.ops.segment_sum(g[src], dst, num_segments=N_NODES)

    g1 = _mm_scale(v, W1, dinv)
    g2 = _combine_mm(scatter(g1), g1, dinv, b1, W2)
    g3 = _combine_mm(scatter(g2), g2, dinv, b2, W3)
    return _combine_final(scatter(g3), g3, dinv, b3)


# speedup vs baseline: 6.5042x; 2.7481x over previous
import functools
import jax
import jax.numpy as jnp
from jax import lax
from jax.experimental import pallas as pl
from jax.experimental.pallas import tpu as pltpu
from jax.experimental.pallas import tpu_sc as plsc

N_NODES = 10000
N_EDGES = 320000
ROW_BLK = 2000

NSC = 2
NSUB = 16
NW = NSC * NSUB
EB = 80
NBLK_TOT = N_EDGES // EB
NBLK = NBLK_TOT // NW
NACC = 10240
STRIPE = NACC // NSUB

_mesh = plsc.VectorSubcoreMesh(core_axis_name="c", subcore_axis_name="s")


def _deg_body(dst_hbm, ones_hbm, zero_hbm, out_hbm,
              idx_d, ones_v, acc, sem):
    c = lax.axis_index("c")
    s = lax.axis_index("s")
    w = c * NSUB + s
    pltpu.sync_copy(ones_hbm, ones_v)
    pltpu.sync_copy(zero_hbm, acc.at[pl.ds(s * STRIPE, STRIPE)])
    plsc.subcore_barrier()

    def body(i, carry):
        blk = w * NBLK + i
        pltpu.sync_copy(dst_hbm.at[blk], idx_d)
        pltpu.sync_copy(ones_v, acc.at[idx_d], add=True)
        return carry

    lax.fori_loop(0, NBLK, body, 0)
    plsc.subcore_barrier()
    row0 = c * NACC + s * STRIPE
    pltpu.sync_copy(acc.at[pl.ds(s * STRIPE, STRIPE)],
                    out_hbm.at[pl.ds(row0, STRIPE)])


_deg_kernel = pl.kernel(
    _deg_body,
    out_type=jax.ShapeDtypeStruct((NSC * NACC, 128), jnp.float32),
    mesh=_mesh,
    scratch_types=[
        pltpu.VMEM((EB,), jnp.int32),
        pltpu.VMEM((EB, 128), jnp.float32),
        pltpu.VMEM_SHARED((NACC, 128), jnp.float32),
        pltpu.SemaphoreType.DMA,
    ],
)


def _scat_body(g_hbm, src_hbm, dst_hbm, zero_hbm, out_hbm,
               idx_s, idx_d, rows, acc, sem):
    c = lax.axis_index("c")
    s = lax.axis_index("s")
    w = c * NSUB + s
    pltpu.sync_copy(zero_hbm, acc.at[pl.ds(s * STRIPE, STRIPE)])
    plsc.subcore_barrier()

    def body(i, carry):
        blk = w * NBLK + i
        pltpu.sync_copy(src_hbm.at[blk], idx_s)
        pltpu.async_copy(g_hbm.at[idx_s], rows, sem).wait()
        pltpu.sync_copy(dst_hbm.at[blk], idx_d)
        pltpu.sync_copy(rows, acc.at[idx_d], add=True)
        return carry

    lax.fori_loop(0, NBLK, body, 0)
    plsc.subcore_barrier()
    row0 = c * NACC + s * STRIPE
    pltpu.sync_copy(acc.at[pl.ds(s * STRIPE, STRIPE)],
                    out_hbm.at[pl.ds(row0, STRIPE)])


_scat_kernel = pl.kernel(
    _scat_body,
    out_type=jax.ShapeDtypeStruct((NSC * NACC, 128), jnp.float32),
    mesh=_mesh,
    scratch_types=[
        pltpu.VMEM((EB,), jnp.int32),
        pltpu.VMEM((EB,), jnp.int32),
        pltpu.VMEM((EB, 128), jnp.float32),
        pltpu.VMEM_SHARED((NACC, 128), jnp.float32),
        pltpu.SemaphoreType.DMA,
    ],
)


def _sc_scatter(g, src2d, dst2d, zero128):
    f = g.shape[1]
    parts = []
    for j in range(f // 128):
        gj = g[:, j * 128:(j + 1) * 128]
        p = _scat_kernel(gj, src2d, dst2d, zero128)
        parts.append((p[:N_NODES], p[NACC:NACC + N_NODES]))
    return parts


def _dinv_body(p0_ref, p1_ref, out_ref):
    out_ref[...] = lax.rsqrt(p0_ref[...] + p1_ref[...] + 1.0)


def _dinv(degp):
    p0 = degp[:N_NODES, :1]
    p1 = degp[NACC:NACC + N_NODES, :1]
    return pl.pallas_call(
        _dinv_body,
        grid=(N_NODES // ROW_BLK,),
        in_specs=[
            pl.BlockSpec((ROW_BLK, 1), lambda i: (i, 0)),
            pl.BlockSpec((ROW_BLK, 1), lambda i: (i, 0)),
        ],
        out_specs=pl.BlockSpec((ROW_BLK, 1), lambda i: (i, 0)),
        out_shape=jax.ShapeDtypeStruct((N_NODES, 1), jnp.float32),
    )(p0, p1)


def _mm_scale_body(x_ref, w_ref, dinv_ref, out_ref):
    h = jnp.dot(x_ref[...], w_ref[...], preferred_element_type=jnp.float32)
    out_ref[...] = h * dinv_ref[...]


def _mm_scale(x, w, dinv):
    n, fin = x.shape
    fout = w.shape[1]
    return pl.pallas_call(
        _mm_scale_body,
        grid=(n // ROW_BLK,),
        in_specs=[
            pl.BlockSpec((ROW_BLK, fin), lambda i: (i, 0)),
            pl.BlockSpec((fin, fout), lambda i: (0, 0)),
            pl.BlockSpec((ROW_BLK, 1), lambda i: (i, 0)),
        ],
        out_specs=pl.BlockSpec((ROW_BLK, fout), lambda i: (i, 0)),
        out_shape=jax.ShapeDtypeStruct((n, fout), jnp.float32),
    )(x, w, dinv)


def _relu_combine(part_refs, g_ref, dinv_ref, b_ref):
    nchunk = len(part_refs) // 2
    cols = []
    for j in range(nchunk):
        p0 = part_refs[2 * j][...]
        p1 = part_refs[2 * j + 1][...]
        gj = g_ref[:, j * 128:(j + 1) * 128]
        bj = b_ref[:, j * 128:(j + 1) * 128]
        cols.append((p0 + p1 + gj) * dinv_ref[...] + bj)
    x = cols[0] if nchunk == 1 else jnp.concatenate(cols, axis=1)
    return jnp.maximum(x, 0.0)


def _combine_mm(parts, g, dinv, b, w):
    n, fin = g.shape
    fout = w.shape[1]
    nchunk = len(parts)

    def body(*refs):
        part_refs = refs[:2 * nchunk]
        g_ref, dinv_ref, b_ref, w_ref, out_ref = refs[2 * nchunk:]
        x = _relu_combine(part_refs, g_ref, dinv_ref, b_ref)
        h = jnp.dot(x, w_ref[...], preferred_element_type=jnp.float32)
        out_ref[...] = h * dinv_ref[...]

    chunk_spec = pl.BlockSpec((ROW_BLK, 128), lambda i: (i, 0))
    in_specs = [chunk_spec] * (2 * nchunk) + [
        pl.BlockSpec((ROW_BLK, fin), lambda i: (i, 0)),
        pl.BlockSpec((ROW_BLK, 1), lambda i: (i, 0)),
        pl.BlockSpec((1, fin), lambda i: (0, 0)),
        pl.BlockSpec((fin, fout), lambda i: (0, 0)),
    ]
    flat = [p for pair in parts for p in pair]
    return pl.pallas_call(
        body,
        grid=(n // ROW_BLK,),
        in_specs=in_specs,
        out_specs=pl.BlockSpec((ROW_BLK, fout), lambda i: (i, 0)),
        out_shape=jax.ShapeDtypeStruct((n, fout), jnp.float32),
    )(*flat, g, dinv, b.reshape(1, fin), w)


def _combine_final(parts, g, dinv, b):
    n, f = g.shape
    nchunk = len(parts)

    def body(*refs):
        part_refs = refs[:2 * nchunk]
        g_ref, dinv_ref, b_ref, out_ref = refs[2 * nchunk:]
        out_ref[...] = _relu_combine(part_refs, g_ref, dinv_ref, b_ref)

    chunk_spec = pl.BlockSpec((ROW_BLK, 128), lambda i: (i, 0))
    in_specs = [chunk_spec] * (2 * nchunk) + [
        pl.BlockSpec((ROW_BLK, f), lambda i: (i, 0)),
        pl.BlockSpec((ROW_BLK, 1), lambda i: (i, 0)),
        pl.BlockSpec((1, f), lambda i: (0, 0)),
    ]
    flat = [p for pair in parts for p in pair]
    return pl.pallas_call(
        body,
        grid=(n // ROW_BLK,),
        in_specs=in_specs,
        out_specs=pl.BlockSpec((ROW_BLK, f), lambda i: (i, 0)),
        out_shape=jax.ShapeDtypeStruct((n, f), jnp.float32),
    )(*flat, g, dinv, b.reshape(1, f))


def kernel(v, edge_index, W1, b1, W2, b2, W3, b3):
    src2d = edge_index[0].reshape(NBLK_TOT, EB)
    dst2d = edge_index[1].reshape(NBLK_TOT, EB)
    ones128 = jnp.ones((EB, 128), jnp.float32)
    zero128 = jnp.zeros((STRIPE, 128), jnp.float32)

    degp = _deg_kernel(dst2d, ones128, zero128)
    dinv = _dinv(degp)

    g1 = _mm_scale(v, W1, dinv)
    g2 = _combine_mm(_sc_scatter(g1, src2d, dst2d, zero128), g1, dinv, b1, W2)
    g3 = _combine_mm(_sc_scatter(g2, src2d, dst2d, zero128), g2, dinv, b2, W3)
    return _combine_final(_sc_scatter(g3, src2d, dst2d, zero128), g3, dinv, b3)
